# Initial kernel scaffold; baseline (speedup 1.0000x reference)
import jax
import jax.numpy as jnp
from jax.experimental import pallas as pl


def kernel(input, weights):
    raise NotImplementedError("write your pallas kernel here")


# speedup vs baseline: 3.2702x; 3.2702x over previous
import jax
import jax.numpy as jnp
from jax.experimental import pallas as pl

_PADDING_IDX = 0
_S_BLK = 512


def _body(in_ref, w_ref, out_ref):
    mask = (in_ref[0, 0][:, None] != _PADDING_IDX)
    out_ref[0] = jnp.where(mask, w_ref[...], 0.0)


def kernel(input, weights):
    bsz, seq_len = input.shape
    dim = weights.shape[1]
    num_s = seq_len // _S_BLK
    inp3 = input.reshape(bsz * num_s, 1, _S_BLK)
    grid = (num_s, bsz)
    return pl.pallas_call(
        _body,
        grid=grid,
        in_specs=[
            pl.BlockSpec((1, 1, _S_BLK), lambda s, b: (b * num_s + s, 0, 0)),
            pl.BlockSpec((_S_BLK, dim), lambda s, b: (s, 0)),
        ],
        out_specs=pl.BlockSpec((1, _S_BLK, dim), lambda s, b: (b, s, 0)),
        out_shape=jax.ShapeDtypeStruct((bsz, seq_len, dim), weights.dtype),
    )(inp3, weights)


# speedup vs baseline: 4.0527x; 1.2393x over previous
import jax
import jax.numpy as jnp
from jax.experimental import pallas as pl

_PADDING_IDX = 0
_S_BLK = 1024


def _body(in_ref, w_ref, out_ref):
    mask = (in_ref[0, 0][:, None] != _PADDING_IDX)
    out_ref[0] = jnp.where(mask, w_ref[...], 0.0)


def kernel(input, weights):
    bsz, seq_len = input.shape
    dim = weights.shape[1]
    num_s = seq_len // _S_BLK
    inp3 = input.reshape(bsz * num_s, 1, _S_BLK)
    grid = (num_s, bsz)
    return pl.pallas_call(
        _body,
        grid=grid,
        in_specs=[
            pl.BlockSpec((1, 1, _S_BLK), lambda s, b: (b * num_s + s, 0, 0)),
            pl.BlockSpec((_S_BLK, dim), lambda s, b: (s, 0)),
        ],
        out_specs=pl.BlockSpec((1, _S_BLK, dim), lambda s, b: (b, s, 0)),
        out_shape=jax.ShapeDtypeStruct((bsz, seq_len, dim), weights.dtype),
    )(inp3, weights)


# speedup vs baseline: 4.5214x; 1.1157x over previous
import jax
import jax.numpy as jnp
from jax.experimental import pallas as pl

_PADDING_IDX = 0
_S_BLK = 2048


def _body(in_ref, w_ref, out_ref):
    mask = (in_ref[0, 0][:, None] != _PADDING_IDX)
    out_ref[0] = jnp.where(mask, w_ref[...], 0.0)


def kernel(input, weights):
    bsz, seq_len = input.shape
    dim = weights.shape[1]
    num_s = seq_len // _S_BLK
    inp3 = input.reshape(bsz * num_s, 1, _S_BLK)
    grid = (num_s, bsz)
    return pl.pallas_call(
        _body,
        grid=grid,
        in_specs=[
            pl.BlockSpec((1, 1, _S_BLK), lambda s, b: (b * num_s + s, 0, 0)),
            pl.BlockSpec((_S_BLK, dim), lambda s, b: (s, 0)),
        ],
        out_specs=pl.BlockSpec((1, _S_BLK, dim), lambda s, b: (b, s, 0)),
        out_shape=jax.ShapeDtypeStruct((bsz, seq_len, dim), weights.dtype),
    )(inp3, weights)


# speedup vs baseline: 4.7516x; 1.0509x over previous
import jax
import jax.numpy as jnp
from jax.experimental import pallas as pl

_PADDING_IDX = 0
_S_BLK = 1024


def _body(in_ref, w_ref, out_ref):
    w = w_ref[...]
    for b in range(out_ref.shape[0]):
        mask = in_ref[:, b:b + 1] != _PADDING_IDX
        out_ref[b] = jnp.where(mask, w, 0.0)


def kernel(input, weights):
    bsz, seq_len = input.shape
    dim = weights.shape[1]
    num_s = seq_len // _S_BLK
    inp_t = input.T
    return pl.pallas_call(
        _body,
        grid=(num_s,),
        in_specs=[
            pl.BlockSpec((_S_BLK, bsz), lambda s: (s, 0)),
            pl.BlockSpec((_S_BLK, dim), lambda s: (s, 0)),
        ],
        out_specs=pl.BlockSpec((bsz, _S_BLK, dim), lambda s: (0, s, 0)),
        out_shape=jax.ShapeDtypeStruct((bsz, seq_len, dim), weights.dtype),
    )(inp_t, weights)
